# baseline (speedup 1.0000x reference)
import functools

import jax
import jax.numpy as jnp
from jax import lax
from jax.experimental import pallas as pl
from jax.experimental.pallas import tpu as pltpu
from jax.experimental.pallas import tpu_sc as plsc

V = 1000000
D = 64
NC = 2
NS = 16
NW = NC * NS
L = 16

NFULL = V // 128
NREM = V - NFULL * 128


def _iota16():
  return lax.iota(jnp.int32, L)


def _full16(x):
  return jnp.full((L,), x, jnp.int32)


def _transpose_block(src, dst, rows, src_col0):
  def body(r, carry):
    for k in range(D // L):
      v = plsc.load_gather(src, [_iota16() + 16 * k, _full16(src_col0 + r)])
      dst[r, pl.ds(16 * k, L)] = v
    return carry

  lax.fori_loop(0, rows, body, None)


@functools.partial(
    pl.kernel,
    mesh=plsc.VectorSubcoreMesh(core_axis_name="c", subcore_axis_name="s"),
    out_type=jax.ShapeDtypeStruct((V, 128), jnp.float32),
    compiler_params=pltpu.CompilerParams(use_tc_tiling_on_sc=True, needs_layout_passes=False),
    scratch_types=[
        pltpu.VMEM((D, 128), jnp.float32),
        pltpu.VMEM((128, 128), jnp.float32),
    ],
)
def _table_rm(t_hbm, rem_hbm, out_hbm, in_v, tr_v):
  wid = lax.axis_index("s") * NC + lax.axis_index("c")

  def body(k, carry):
    j = wid + k * NW

    @pl.when(j < NFULL)
    def _():
      pltpu.sync_copy(t_hbm.at[:, pl.ds(j * 128, 128)], in_v)
      _transpose_block(in_v, tr_v, 128, 0)
      pltpu.sync_copy(tr_v, out_hbm.at[pl.ds(j * 128, 128)])

    return carry

  lax.fori_loop(0, NFULL // NW + 1, body, None)

  @pl.when(wid == NW - 1)
  def _():
    pltpu.sync_copy(rem_hbm, in_v.at[pl.ds(0, NREM)])
    pltpu.sync_copy(in_v.at[pl.ds(0, NREM)],
                    out_hbm.at[pl.ds(NFULL * 128, NREM)])


def _make_gather(b: int, t: int):
  jb = b // NW

  @functools.partial(
      pl.kernel,
      mesh=plsc.VectorSubcoreMesh(core_axis_name="c", subcore_axis_name="s"),
      out_type=jax.ShapeDtypeStruct((t, D, b), jnp.float32),
      compiler_params=pltpu.CompilerParams(use_tc_tiling_on_sc=True, needs_layout_passes=False),
      scratch_types=[
          pltpu.VMEM((t, jb), jnp.int32),
          pltpu.VMEM((jb, 128), jnp.float32),
          pltpu.VMEM((D, jb), jnp.float32),
          pltpu.SemaphoreType.DMA,
      ],
  )
  def gather_kernel(table_hbm, idx_hbm, out_hbm, idx_v, rows_v, tr_v, gsem):
    wid = lax.axis_index("s") * NC + lax.axis_index("c")
    pltpu.sync_copy(idx_hbm.at[:, pl.ds(wid * jb, jb)], idx_v)

    def body(tt, carry):
      pltpu.async_copy(table_hbm.at[idx_v.at[tt]], rows_v, gsem).wait()
      def trb(d, c2):
        for m in range(jb // L):
          v = plsc.load_gather(rows_v, [_iota16() + 16 * m, _full16(d)])
          tr_v[d, pl.ds(16 * m, L)] = v
        return c2

      lax.fori_loop(0, D, trb, None)
      pltpu.sync_copy(tr_v, out_hbm.at[tt, :, pl.ds(wid * jb, jb)])
      return carry

    lax.fori_loop(0, t, body, None)

  return gather_kernel


def kernel(token_ids, embeddings):
  b, t = token_ids.shape
  t_feat_major = embeddings.T
  idx_t = token_ids.T.astype(jnp.int32)
  rem_pad = jnp.pad(embeddings[V - NREM:], ((0, 0), (0, 128 - D)))
  table_rm = _table_rm(t_feat_major, rem_pad)
  out_t = _make_gather(b, t)(table_rm, idx_t)
  return jnp.transpose(out_t, (2, 0, 1))


# speedup vs baseline: 1.5878x; 1.5878x over previous
import functools

import jax
import jax.numpy as jnp
from jax import lax
from jax.experimental import pallas as pl
from jax.experimental.pallas import tpu as pltpu
from jax.experimental.pallas import tpu_sc as plsc

V = 1000000
D = 64
NC = 2
NS = 16
NW = NC * NS
L = 16

NFULL = V // 128
NREM = V - NFULL * 128
NK1 = NFULL // NW + 1


def _full16(x):
  return jnp.full((L,), x, jnp.int32)


def _row_ids(m):
  return lax.iota(jnp.int32, L) + L * m


def _transpose_into(src, dst, n_src_rows):
  n_dst_vregs = len(range(0, dst.shape[0], L))

  def body(r, carry):
    rcol = _full16(r)
    for m in range(n_dst_vregs):
      v = src[r, pl.ds(L * m, L)]
      plsc.store_scatter(dst, [_row_ids(m), rcol], v)
    return carry

  lax.fori_loop(0, n_src_rows, body, None)


@functools.partial(
    pl.kernel,
    mesh=plsc.VectorSubcoreMesh(core_axis_name="c", subcore_axis_name="s"),
    out_type=jax.ShapeDtypeStruct((V, 128), jnp.float32),
    compiler_params=pltpu.CompilerParams(
        use_tc_tiling_on_sc=True, needs_layout_passes=False),
    scratch_types=[
        pltpu.VMEM((D, 128), jnp.float32),
        pltpu.VMEM((D, 128), jnp.float32),
        pltpu.VMEM((128, 128), jnp.float32),
        pltpu.VMEM((128, 128), jnp.float32),
        pltpu.SemaphoreType.DMA,
        pltpu.SemaphoreType.DMA,
        pltpu.SemaphoreType.DMA,
        pltpu.SemaphoreType.DMA,
    ],
)
def _table_rm(t_hbm, rem_hbm, out_hbm, in0, in1, tr0, tr1, isem0, isem1,
              osem0, osem1):
  wid = lax.axis_index("s") * NC + lax.axis_index("c")
  ins = (in0, in1)
  trs = (tr0, tr1)
  isems = (isem0, isem1)
  osems = (osem0, osem1)

  def fire_in(k, s):
    j = wid + k * NW

    @pl.when(j < NFULL)
    def _():
      pltpu.async_copy(t_hbm.at[:, pl.ds(j * 128, 128)], ins[s], isems[s])

  fire_in(0, 0)

  def pair_body(p, carry):
    for s in range(2):
      k = 2 * p + s
      j = wid + k * NW
      fire_in(k + 1, 1 - s)

      @pl.when(j < NFULL)
      def _():
        pltpu.make_async_copy(
            t_hbm.at[:, pl.ds(0, 128)], ins[s], isems[s]).wait()

        @pl.when(k >= 2)
        def _():
          pltpu.make_async_copy(
              trs[s], out_hbm.at[pl.ds(0, 128)], osems[s]).wait()

        _transpose_into(ins[s], trs[s], D)
        pltpu.async_copy(trs[s], out_hbm.at[pl.ds(j * 128, 128)], osems[s])

    return carry

  lax.fori_loop(0, (NK1 + 1) // 2, pair_body, None)
  for s in range(2):
    pltpu.make_async_copy(trs[s], out_hbm.at[pl.ds(0, 128)], osems[s]).wait()

  @pl.when(wid == NW - 1)
  def _():
    pltpu.sync_copy(rem_hbm, in0)
    pltpu.sync_copy(in0, out_hbm.at[pl.ds(NFULL * 128, NREM)])


def _make_gather(b: int, t: int):
  jb = b // NW

  @functools.partial(
      pl.kernel,
      mesh=plsc.VectorSubcoreMesh(core_axis_name="c", subcore_axis_name="s"),
      out_type=jax.ShapeDtypeStruct((t, D, b), jnp.float32),
      compiler_params=pltpu.CompilerParams(
          use_tc_tiling_on_sc=True, needs_layout_passes=False),
      scratch_types=[
          pltpu.VMEM((t, jb), jnp.int32),
          pltpu.VMEM((jb, 128), jnp.float32),
          pltpu.VMEM((jb, 128), jnp.float32),
          pltpu.VMEM((D, jb), jnp.float32),
          pltpu.VMEM((D, jb), jnp.float32),
          pltpu.SemaphoreType.DMA,
          pltpu.SemaphoreType.DMA,
          pltpu.SemaphoreType.DMA,
          pltpu.SemaphoreType.DMA,
      ],
  )
  def gather_kernel(table_hbm, idx_hbm, out_hbm, idx_v, rows0, rows1, tr0,
                    tr1, gsem0, gsem1, osem0, osem1):
    wid = lax.axis_index("s") * NC + lax.axis_index("c")
    rows = (rows0, rows1)
    trs = (tr0, tr1)
    gsems = (gsem0, gsem1)
    osems = (osem0, osem1)
    col0 = wid * jb
    pltpu.sync_copy(idx_hbm.at[:, pl.ds(col0, jb)], idx_v)

    def fire_gather(tt, s):
      pltpu.async_copy(table_hbm.at[idx_v.at[tt]], rows[s], gsems[s])

    fire_gather(0, 0)

    def pair_body(p, carry):
      for s in range(2):
        tt = 2 * p + s

        @pl.when(tt + 1 < t)
        def _():
          fire_gather(tt + 1, 1 - s)

        pltpu.make_async_copy(
            table_hbm.at[idx_v.at[0]], rows[s], gsems[s]).wait()

        @pl.when(tt >= 2)
        def _():
          pltpu.make_async_copy(
              trs[s], out_hbm.at[0, :, pl.ds(col0, jb)], osems[s]).wait()

        _transpose_into(rows[s], trs[s], jb)
        pltpu.async_copy(trs[s], out_hbm.at[tt, :, pl.ds(col0, jb)], osems[s])
      return carry

    lax.fori_loop(0, t // 2, pair_body, None)
    for s in range(2):
      pltpu.make_async_copy(
          trs[s], out_hbm.at[0, :, pl.ds(col0, jb)], osems[s]).wait()

  return gather_kernel


def kernel(token_ids, embeddings):
  b, t = token_ids.shape
  t_feat_major = embeddings.T
  idx_t = token_ids.T.astype(jnp.int32)
  rem_pad = jnp.pad(embeddings[V - NREM:], ((0, 0), (0, 128 - D)))
  table_rm = _table_rm(t_feat_major, rem_pad)
  out_t = _make_gather(b, t)(table_rm, idx_t)
  return jnp.transpose(out_t, (2, 0, 1))


# speedup vs baseline: 1.5933x; 1.0035x over previous
import functools

import jax
import jax.numpy as jnp
from jax import lax
from jax.experimental import pallas as pl
from jax.experimental.pallas import tpu as pltpu
from jax.experimental.pallas import tpu_sc as plsc

V = 1000000
D = 64
NC = 2
NS = 16
NW = NC * NS
L = 16

NFULL = V // 128
NREM = V - NFULL * 128
NK1 = NFULL // NW + 1


def _full16(x):
  return jnp.full((L,), x, jnp.int32)


def _transpose_into(src, dst, n_src_rows):
  n_dst_vregs = dst.shape[0] // L
  row_ids = [lax.iota(jnp.int32, L) + L * m for m in range(n_dst_vregs)]

  def body(r2, carry):
    for u in range(2):
      r = r2 * 2 + u
      rcol = _full16(r)
      for m in range(n_dst_vregs):
        v = src[r, pl.ds(L * m, L)]
        plsc.store_scatter(dst, [row_ids[m], rcol], v)
    return carry

  lax.fori_loop(0, n_src_rows // 2, body, None)


@functools.partial(
    pl.kernel,
    mesh=plsc.VectorSubcoreMesh(core_axis_name="c", subcore_axis_name="s"),
    out_type=jax.ShapeDtypeStruct((V, 128), jnp.float32),
    compiler_params=pltpu.CompilerParams(
        use_tc_tiling_on_sc=True, needs_layout_passes=False, disable_bounds_checks=True),
    scratch_types=[
        pltpu.VMEM((D, 128), jnp.float32),
        pltpu.VMEM((D, 128), jnp.float32),
        pltpu.VMEM((128, 128), jnp.float32),
        pltpu.VMEM((128, 128), jnp.float32),
        pltpu.SemaphoreType.DMA,
        pltpu.SemaphoreType.DMA,
        pltpu.SemaphoreType.DMA,
        pltpu.SemaphoreType.DMA,
    ],
)
def _table_rm(t_hbm, rem_hbm, out_hbm, in0, in1, tr0, tr1, isem0, isem1,
              osem0, osem1):
  wid = lax.axis_index("s") * NC + lax.axis_index("c")
  ins = (in0, in1)
  trs = (tr0, tr1)
  isems = (isem0, isem1)
  osems = (osem0, osem1)

  def fire_in(k, s):
    j = wid + k * NW

    @pl.when(j < NFULL)
    def _():
      pltpu.async_copy(t_hbm.at[:, pl.ds(j * 128, 128)], ins[s], isems[s])

  fire_in(0, 0)

  def pair_body(p, carry):
    for s in range(2):
      k = 2 * p + s
      j = wid + k * NW
      fire_in(k + 1, 1 - s)

      @pl.when(j < NFULL)
      def _():
        pltpu.make_async_copy(
            t_hbm.at[:, pl.ds(0, 128)], ins[s], isems[s]).wait()

        @pl.when(k >= 2)
        def _():
          pltpu.make_async_copy(
              trs[s], out_hbm.at[pl.ds(0, 128)], osems[s]).wait()

        _transpose_into(ins[s], trs[s], D)
        pltpu.async_copy(trs[s], out_hbm.at[pl.ds(j * 128, 128)], osems[s])

    return carry

  lax.fori_loop(0, (NK1 + 1) // 2, pair_body, None)
  for s in range(2):
    pltpu.make_async_copy(trs[s], out_hbm.at[pl.ds(0, 128)], osems[s]).wait()

  @pl.when(wid == NW - 1)
  def _():
    pltpu.sync_copy(rem_hbm, in0)
    pltpu.sync_copy(in0, out_hbm.at[pl.ds(NFULL * 128, NREM)])


def _make_gather(b: int, t: int):
  jb = b // NW

  @functools.partial(
      pl.kernel,
      mesh=plsc.VectorSubcoreMesh(core_axis_name="c", subcore_axis_name="s"),
      out_type=jax.ShapeDtypeStruct((t, D, b), jnp.float32),
      compiler_params=pltpu.CompilerParams(
          use_tc_tiling_on_sc=True, needs_layout_passes=False, disable_bounds_checks=True),
      scratch_types=[
          pltpu.VMEM((t, jb), jnp.int32),
          pltpu.VMEM((jb, 128), jnp.float32),
          pltpu.VMEM((jb, 128), jnp.float32),
          pltpu.VMEM((D, jb), jnp.float32),
          pltpu.VMEM((D, jb), jnp.float32),
          pltpu.SemaphoreType.DMA,
          pltpu.SemaphoreType.DMA,
          pltpu.SemaphoreType.DMA,
          pltpu.SemaphoreType.DMA,
      ],
  )
  def gather_kernel(table_hbm, idx_hbm, out_hbm, idx_v, rows0, rows1, tr0,
                    tr1, gsem0, gsem1, osem0, osem1):
    wid = lax.axis_index("s") * NC + lax.axis_index("c")
    rows = (rows0, rows1)
    trs = (tr0, tr1)
    gsems = (gsem0, gsem1)
    osems = (osem0, osem1)
    col0 = wid * jb
    pltpu.sync_copy(idx_hbm.at[:, pl.ds(col0, jb)], idx_v)

    def fire_gather(tt, s):
      pltpu.async_copy(table_hbm.at[idx_v.at[tt]], rows[s], gsems[s])

    fire_gather(0, 0)

    def pair_body(p, carry):
      for s in range(2):
        tt = 2 * p + s

        @pl.when(tt + 1 < t)
        def _():
          fire_gather(tt + 1, 1 - s)

        pltpu.make_async_copy(
            table_hbm.at[idx_v.at[0]], rows[s], gsems[s]).wait()

        @pl.when(tt >= 2)
        def _():
          pltpu.make_async_copy(
              trs[s], out_hbm.at[0, :, pl.ds(col0, jb)], osems[s]).wait()

        _transpose_into(rows[s], trs[s], jb)
        pltpu.async_copy(trs[s], out_hbm.at[tt, :, pl.ds(col0, jb)], osems[s])
      return carry

    lax.fori_loop(0, t // 2, pair_body, None)
    for s in range(2):
      pltpu.make_async_copy(
          trs[s], out_hbm.at[0, :, pl.ds(col0, jb)], osems[s]).wait()

  return gather_kernel


def kernel(token_ids, embeddings):
  b, t = token_ids.shape
  t_feat_major = embeddings.T
  idx_t = token_ids.T.astype(jnp.int32)
  rem_pad = jnp.pad(embeddings[V - NREM:], ((0, 0), (0, 128 - D)))
  table_rm = _table_rm(t_feat_major, rem_pad)
  out_t = _make_gather(b, t)(table_rm, idx_t)
  return jnp.transpose(out_t, (2, 0, 1))
